# baseline (speedup 1.0000x reference)
import jax
import jax.numpy as jnp
from jax.experimental import pallas as pl

B, DIM, H, W = 16, 384, 32, 32
NE, TOPK, R, ALPHA = 3, 2, 8, 8
HID = 4 * DIM
EPS = 1e-6
PADHW = 40
TP = H * W
LW = 32
SCALING = ALPHA / R


def _gelu(x):
    return 0.5 * x * (1.0 + jax.lax.erf(x * 0.7071067811865476))


def _block_kernel(xpad_ref, cw_ref, cb_ref, effw_ref, effb_ref,
                  fc1_ref, fc1b_ref, fc2_ref, fc2b_ref,
                  wd_ref, wu_ref, i0_ref, i1_ref, p0_ref, p1_ref,
                  gamma_ref, out_ref):
    xp = xpad_ref[0]

    acc = jnp.zeros((H, W, DIM), jnp.float32)
    for dh in range(7):
        for dw in range(7):
            acc = acc + xp[dh:dh + H, dw:dw + W, :] * cw_ref[dh * 7 + dw]
    acc = acc + cb_ref[0]

    xt = acc.reshape(TP, DIM)

    mu = jnp.mean(xt, axis=-1, keepdims=True)
    var = jnp.mean((xt - mu) ** 2, axis=-1, keepdims=True)
    xn = (xt - mu) * jax.lax.rsqrt(var + EPS) * effw_ref[0] + effb_ref[0]

    xnb = xn.astype(jnp.bfloat16)
    h = _gelu(jnp.dot(xnb, fc1_ref[...], preferred_element_type=jnp.float32)
              + fc1b_ref[0])
    y = (jnp.dot(h.astype(jnp.bfloat16), fc2_ref[...],
                 preferred_element_type=jnp.float32) + fc2b_ref[0])

    g = _gelu(jnp.dot(xnb, wd_ref[...], preferred_element_type=jnp.float32))
    e = jax.lax.broadcasted_iota(jnp.int32, (TP, LW), 1) // R
    i0 = i0_ref[0]
    i1 = i1_ref[0]
    s = (jnp.where(e == i0, p0_ref[0], 0.0)
         + jnp.where(e == i1, p1_ref[0], 0.0))
    moe = jnp.dot((g * s).astype(jnp.bfloat16), wu_ref[...],
                  preferred_element_type=jnp.float32) * SCALING

    xc = xp[3:3 + H, 3:3 + W, :].reshape(TP, DIM)
    out_ref[0] = xc + gamma_ref[0] * (y + moe)


def kernel(x, conv_w, conv_b, norm_w, norm_b, dom_norm_w, dom_norm_b,
           fc1_w, fc1_b, fc2_w, fc2_b, w_down, w_up, gamma,
           gate_probs, topk_probs, routing_weights, topk_indices):
    xt = jnp.transpose(x, (0, 2, 3, 1))
    xpad = jnp.pad(xt, ((0, 0), (3, PADHW - 3 - H), (3, PADHW - 3 - W), (0, 0)))
    cw = conv_w.reshape(DIM, 49).T
    effw = (norm_w[None, :] + routing_weights @ dom_norm_w).reshape(B, 1, DIM)
    effb = (norm_b[None, :] + routing_weights @ dom_norm_b).reshape(B, 1, DIM)
    wd = jnp.transpose(w_down, (1, 0, 2)).reshape(DIM, NE * R)
    wd = jnp.pad(wd, ((0, 0), (0, LW - NE * R)))
    wu = jnp.pad(w_up.reshape(NE * R, DIM), ((0, LW - NE * R), (0, 0)))
    i0 = topk_indices[:, 0].reshape(B, TP, 1)
    i1 = topk_indices[:, 1].reshape(B, TP, 1)
    p0 = topk_probs[:, 0].reshape(B, TP, 1)
    p1 = topk_probs[:, 1].reshape(B, TP, 1)
    gam = gamma.reshape(1, DIM)

    full = lambda shape: pl.BlockSpec(shape, lambda b: (0,) * len(shape))
    out = pl.pallas_call(
        _block_kernel,
        grid=(B,),
        in_specs=[
            pl.BlockSpec((1, PADHW, PADHW, DIM), lambda b: (b, 0, 0, 0)),
            full((49, DIM)),
            full((1, DIM)),
            pl.BlockSpec((1, 1, DIM), lambda b: (b, 0, 0)),
            pl.BlockSpec((1, 1, DIM), lambda b: (b, 0, 0)),
            full((DIM, HID)),
            full((1, HID)),
            full((HID, DIM)),
            full((1, DIM)),
            full((DIM, LW)),
            full((LW, DIM)),
            pl.BlockSpec((1, TP, 1), lambda b: (b, 0, 0)),
            pl.BlockSpec((1, TP, 1), lambda b: (b, 0, 0)),
            pl.BlockSpec((1, TP, 1), lambda b: (b, 0, 0)),
            pl.BlockSpec((1, TP, 1), lambda b: (b, 0, 0)),
            full((1, DIM)),
        ],
        out_specs=pl.BlockSpec((1, TP, DIM), lambda b: (b, 0, 0)),
        out_shape=jax.ShapeDtypeStruct((B, TP, DIM), jnp.float32),
    )(xpad, cw, conv_b.reshape(1, DIM), effw, effb,
      fc1_w.astype(jnp.bfloat16), fc1_b.reshape(1, HID),
      fc2_w.astype(jnp.bfloat16), fc2_b.reshape(1, DIM),
      wd.astype(jnp.bfloat16), wu.astype(jnp.bfloat16), i0, i1, p0, p1, gam)

    return jnp.transpose(out.reshape(B, H, W, DIM), (0, 3, 1, 2))


# speedup vs baseline: 1.2870x; 1.2870x over previous
import jax
import jax.numpy as jnp
from jax.experimental import pallas as pl

B, DIM, H, W = 16, 384, 32, 32
NE, TOPK, R, ALPHA = 3, 2, 8, 8
HID = 4 * DIM
EPS = 1e-6
PADHW = 40
TP = H * W
LW = 32
SCALING = ALPHA / R


def _gelu(x):
    return 0.5 * x * (1.0 + jax.lax.erf(x * 0.7071067811865476))


def _block_kernel(xpad_ref, cw_ref, cb_ref, effw_ref, effb_ref,
                  fc1_ref, fc1b_ref, fc2_ref, fc2b_ref,
                  wd_ref, wu_ref, i0_ref, i1_ref, p0_ref, p1_ref,
                  gamma_ref, out_ref):
    xp = xpad_ref[0]

    acc = jnp.zeros((H, W, DIM), jnp.float32)
    for dw in range(7):
        a = xp[0:H, :, :] * cw_ref[dw]
        for dh in range(1, 7):
            a = a + xp[dh:dh + H, :, :] * cw_ref[dh * 7 + dw]
        acc = acc + a[:, dw:dw + W, :]
    acc = acc + cb_ref[0]

    xt = acc.reshape(TP, DIM)

    mu = jnp.mean(xt, axis=-1, keepdims=True)
    var = jnp.mean((xt - mu) ** 2, axis=-1, keepdims=True)
    xn = (xt - mu) * jax.lax.rsqrt(var + EPS) * effw_ref[0] + effb_ref[0]

    xnb = xn.astype(jnp.bfloat16)
    h = _gelu(jnp.dot(xnb, fc1_ref[...], preferred_element_type=jnp.float32)
              + fc1b_ref[0])
    y = (jnp.dot(h.astype(jnp.bfloat16), fc2_ref[...],
                 preferred_element_type=jnp.float32) + fc2b_ref[0])

    g = _gelu(jnp.dot(xnb, wd_ref[...], preferred_element_type=jnp.float32))
    e = jax.lax.broadcasted_iota(jnp.int32, (TP, LW), 1) // R
    i0 = i0_ref[0]
    i1 = i1_ref[0]
    s = (jnp.where(e == i0, p0_ref[0], 0.0)
         + jnp.where(e == i1, p1_ref[0], 0.0))
    moe = jnp.dot((g * s).astype(jnp.bfloat16), wu_ref[...],
                  preferred_element_type=jnp.float32) * SCALING

    xc = xp[3:3 + H, 3:3 + W, :].reshape(TP, DIM)
    out_ref[0] = xc + gamma_ref[0] * (y + moe)


def kernel(x, conv_w, conv_b, norm_w, norm_b, dom_norm_w, dom_norm_b,
           fc1_w, fc1_b, fc2_w, fc2_b, w_down, w_up, gamma,
           gate_probs, topk_probs, routing_weights, topk_indices):
    xt = jnp.transpose(x, (0, 2, 3, 1))
    xpad = jnp.pad(xt, ((0, 0), (3, PADHW - 3 - H), (3, PADHW - 3 - W), (0, 0)))
    cw = conv_w.reshape(DIM, 49).T
    effw = (norm_w[None, :] + routing_weights @ dom_norm_w).reshape(B, 1, DIM)
    effb = (norm_b[None, :] + routing_weights @ dom_norm_b).reshape(B, 1, DIM)
    wd = jnp.transpose(w_down, (1, 0, 2)).reshape(DIM, NE * R)
    wd = jnp.pad(wd, ((0, 0), (0, LW - NE * R)))
    wu = jnp.pad(w_up.reshape(NE * R, DIM), ((0, LW - NE * R), (0, 0)))
    i0 = topk_indices[:, 0].reshape(B, TP, 1)
    i1 = topk_indices[:, 1].reshape(B, TP, 1)
    p0 = topk_probs[:, 0].reshape(B, TP, 1)
    p1 = topk_probs[:, 1].reshape(B, TP, 1)
    gam = gamma.reshape(1, DIM)

    full = lambda shape: pl.BlockSpec(shape, lambda b: (0,) * len(shape))
    out = pl.pallas_call(
        _block_kernel,
        grid=(B,),
        in_specs=[
            pl.BlockSpec((1, PADHW, PADHW, DIM), lambda b: (b, 0, 0, 0)),
            full((49, DIM)),
            full((1, DIM)),
            pl.BlockSpec((1, 1, DIM), lambda b: (b, 0, 0)),
            pl.BlockSpec((1, 1, DIM), lambda b: (b, 0, 0)),
            full((DIM, HID)),
            full((1, HID)),
            full((HID, DIM)),
            full((1, DIM)),
            full((DIM, LW)),
            full((LW, DIM)),
            pl.BlockSpec((1, TP, 1), lambda b: (b, 0, 0)),
            pl.BlockSpec((1, TP, 1), lambda b: (b, 0, 0)),
            pl.BlockSpec((1, TP, 1), lambda b: (b, 0, 0)),
            pl.BlockSpec((1, TP, 1), lambda b: (b, 0, 0)),
            full((1, DIM)),
        ],
        out_specs=pl.BlockSpec((1, TP, DIM), lambda b: (b, 0, 0)),
        out_shape=jax.ShapeDtypeStruct((B, TP, DIM), jnp.float32),
    )(xpad, cw, conv_b.reshape(1, DIM), effw, effb,
      fc1_w.astype(jnp.bfloat16), fc1_b.reshape(1, HID),
      fc2_w.astype(jnp.bfloat16), fc2_b.reshape(1, DIM),
      wd.astype(jnp.bfloat16), wu.astype(jnp.bfloat16), i0, i1, p0, p1, gam)

    return jnp.transpose(out.reshape(B, H, W, DIM), (0, 3, 1, 2))


# speedup vs baseline: 1.5088x; 1.1724x over previous
import jax
import jax.numpy as jnp
from jax.experimental import pallas as pl

B, DIM, H, W = 16, 384, 32, 32
NE, TOPK, R, ALPHA = 3, 2, 8, 8
HID = 4 * DIM
EPS = 1e-6
PADHW = 40
TP = H * W
LW = 32
SCALING = ALPHA / R


def _gelu(x):
    return 0.5 * x * (1.0 + jax.lax.erf(x * 0.7071067811865476))


def _block_kernel(xpad_ref, cw_ref, cb_ref, effw_ref, effb_ref,
                  fc1_ref, fc1b_ref, fc2_ref, fc2b_ref,
                  wd_ref, wu_ref, i0_ref, i1_ref, p0_ref, p1_ref,
                  gamma_ref, out_ref):
    xp = xpad_ref[0]

    xpb = xp.astype(jnp.bfloat16)
    cwb = cw_ref[...].astype(jnp.bfloat16)
    acc = jnp.zeros((H, W, DIM), jnp.float32)
    for dw in range(7):
        a = xpb[0:H, :, :] * cwb[dw]
        for dh in range(1, 7):
            a = a + xpb[dh:dh + H, :, :] * cwb[dh * 7 + dw]
        acc = acc + a[:, dw:dw + W, :].astype(jnp.float32)
    acc = acc + cb_ref[0]

    xt = acc.reshape(TP, DIM)

    mu = jnp.mean(xt, axis=-1, keepdims=True)
    var = jnp.mean((xt - mu) ** 2, axis=-1, keepdims=True)
    xn = (xt - mu) * jax.lax.rsqrt(var + EPS) * effw_ref[0] + effb_ref[0]

    xnb = xn.astype(jnp.bfloat16)
    h = _gelu(jnp.dot(xnb, fc1_ref[...], preferred_element_type=jnp.float32)
              + fc1b_ref[0])
    y = (jnp.dot(h.astype(jnp.bfloat16), fc2_ref[...],
                 preferred_element_type=jnp.float32) + fc2b_ref[0])

    g = _gelu(jnp.dot(xnb, wd_ref[...], preferred_element_type=jnp.float32))
    e = jax.lax.broadcasted_iota(jnp.int32, (TP, LW), 1) // R
    i0 = i0_ref[0]
    i1 = i1_ref[0]
    s = (jnp.where(e == i0, p0_ref[0], 0.0)
         + jnp.where(e == i1, p1_ref[0], 0.0))
    moe = jnp.dot((g * s).astype(jnp.bfloat16), wu_ref[...],
                  preferred_element_type=jnp.float32) * SCALING

    xc = xp[3:3 + H, 3:3 + W, :].reshape(TP, DIM)
    out_ref[0] = xc + gamma_ref[0] * (y + moe)


def kernel(x, conv_w, conv_b, norm_w, norm_b, dom_norm_w, dom_norm_b,
           fc1_w, fc1_b, fc2_w, fc2_b, w_down, w_up, gamma,
           gate_probs, topk_probs, routing_weights, topk_indices):
    xt = jnp.transpose(x, (0, 2, 3, 1))
    xpad = jnp.pad(xt, ((0, 0), (3, PADHW - 3 - H), (3, PADHW - 3 - W), (0, 0)))
    cw = conv_w.reshape(DIM, 49).T
    effw = (norm_w[None, :] + routing_weights @ dom_norm_w).reshape(B, 1, DIM)
    effb = (norm_b[None, :] + routing_weights @ dom_norm_b).reshape(B, 1, DIM)
    wd = jnp.transpose(w_down, (1, 0, 2)).reshape(DIM, NE * R)
    wd = jnp.pad(wd, ((0, 0), (0, LW - NE * R)))
    wu = jnp.pad(w_up.reshape(NE * R, DIM), ((0, LW - NE * R), (0, 0)))
    i0 = topk_indices[:, 0].reshape(B, TP, 1)
    i1 = topk_indices[:, 1].reshape(B, TP, 1)
    p0 = topk_probs[:, 0].reshape(B, TP, 1)
    p1 = topk_probs[:, 1].reshape(B, TP, 1)
    gam = gamma.reshape(1, DIM)

    full = lambda shape: pl.BlockSpec(shape, lambda b: (0,) * len(shape))
    out = pl.pallas_call(
        _block_kernel,
        grid=(B,),
        in_specs=[
            pl.BlockSpec((1, PADHW, PADHW, DIM), lambda b: (b, 0, 0, 0)),
            full((49, DIM)),
            full((1, DIM)),
            pl.BlockSpec((1, 1, DIM), lambda b: (b, 0, 0)),
            pl.BlockSpec((1, 1, DIM), lambda b: (b, 0, 0)),
            full((DIM, HID)),
            full((1, HID)),
            full((HID, DIM)),
            full((1, DIM)),
            full((DIM, LW)),
            full((LW, DIM)),
            pl.BlockSpec((1, TP, 1), lambda b: (b, 0, 0)),
            pl.BlockSpec((1, TP, 1), lambda b: (b, 0, 0)),
            pl.BlockSpec((1, TP, 1), lambda b: (b, 0, 0)),
            pl.BlockSpec((1, TP, 1), lambda b: (b, 0, 0)),
            full((1, DIM)),
        ],
        out_specs=pl.BlockSpec((1, TP, DIM), lambda b: (b, 0, 0)),
        out_shape=jax.ShapeDtypeStruct((B, TP, DIM), jnp.float32),
    )(xpad, cw, conv_b.reshape(1, DIM), effw, effb,
      fc1_w.astype(jnp.bfloat16), fc1_b.reshape(1, HID),
      fc2_w.astype(jnp.bfloat16), fc2_b.reshape(1, DIM),
      wd.astype(jnp.bfloat16), wu.astype(jnp.bfloat16), i0, i1, p0, p1, gam)

    return jnp.transpose(out.reshape(B, H, W, DIM), (0, 3, 1, 2))
